# baseline (speedup 1.0000x reference)
import functools

import jax
import jax.numpy as jnp
from jax import lax
from jax.experimental import pallas as pl
from jax.experimental.pallas import tpu as pltpu
from jax.experimental.pallas import tpu_sc as plsc

N_NODES = 10000
N_EDGES = 320000
D_FEAT = 128
LANES = 16
NUM_WORKERS = 32
EPW = N_EDGES // NUM_WORKERS
CHUNK = 80
NCHUNKS = EPW // CHUNK
GROUPS = CHUNK // LANES

N_PAD = 10240
M_BLK = 512
M_TILES = N_PAD // M_BLK
N_TILES = N_PAD // D_FEAT
BLK_WORDS = M_BLK * D_FEAT
G_WORDS = M_TILES * N_TILES * BLK_WORDS


def _matmul_body(x_ref, y_ref, o_ref):
    r = jnp.dot(x_ref[...], y_ref[...], preferred_element_type=jnp.float32)
    o_ref[...] = r.reshape(BLK_WORDS)


@jax.jit
def _gram(x, y):
    return pl.pallas_call(
        _matmul_body,
        grid=(M_TILES, N_TILES),
        in_specs=[
            pl.BlockSpec((M_BLK, D_FEAT), lambda m, t: (m, 0)),
            pl.BlockSpec((D_FEAT, D_FEAT), lambda m, t: (0, t)),
        ],
        out_specs=pl.BlockSpec((BLK_WORDS,), lambda m, t: (t * M_TILES + m,)),
        out_shape=jax.ShapeDtypeStruct((G_WORDS,), jnp.float32),
    )(x, y)


def _gather_body(g_hbm, src_hbm, dst_hbm, out_hbm,
                 idx_s, idx_d, w0, w1, out_v, sem0, sem1):
    c = lax.axis_index("c")
    s = lax.axis_index("s")
    wid = s * 2 + c
    base = wid * EPW

    pltpu.sync_copy(src_hbm.at[pl.ds(base, EPW)], idx_s)
    pltpu.sync_copy(dst_hbm.at[pl.ds(base, EPW)], idx_d)

    bufs = ((w0, sem0), (w1, sem1))

    def prep(g, wb):
        off = g * CHUNK
        for k in range(GROUPS):
            sv = idx_s[pl.ds(off + k * LANES, LANES)]
            dv = idx_d[pl.ds(off + k * LANES, LANES)]
            blk = (dv >> 7) * M_TILES + (sv >> 9)
            w = (blk << 16) | ((sv & 511) << 7) | (dv & 127)
            wb[pl.ds(k * LANES, LANES)] = w

    def fire(g, wb, sem):
        pltpu.async_copy(g_hbm.at[wb.at[pl.ds(0, CHUNK)]],
                         out_v.at[pl.ds(g * CHUNK, CHUNK)], sem)

    def drain(sem):
        pltpu.make_async_copy(g_hbm.at[pl.ds(0, CHUNK)],
                              out_v.at[pl.ds(0, CHUNK)], sem).wait()

    prep(0, w0)
    fire(0, w0, sem0)
    prep(1, w1)
    fire(1, w1, sem1)

    def outer(g2, carry):
        for b in range(2):
            g = g2 * 2 + b
            wb, sem = bufs[b]
            drain(sem)

            @pl.when(g + 2 < NCHUNKS)
            def _():
                prep(g + 2, wb)
                fire(g + 2, wb, sem)
        return carry

    lax.fori_loop(0, (NCHUNKS - 1) // 2, outer, 0)
    drain(sem0)

    pltpu.sync_copy(out_v, out_hbm.at[pl.ds(base, EPW)])


@jax.jit
def _gather_flat(g, src, dst):
    mesh = plsc.VectorSubcoreMesh(core_axis_name="c", subcore_axis_name="s")
    kern = functools.partial(
        pl.kernel,
        mesh=mesh,
        compiler_params=pltpu.CompilerParams(needs_layout_passes=False,
                                             use_tc_tiling_on_sc=False),
        out_type=jax.ShapeDtypeStruct((N_EDGES,), jnp.float32),
        scratch_types=[
            pltpu.VMEM((EPW,), jnp.int32),
            pltpu.VMEM((EPW,), jnp.int32),
            pltpu.VMEM((CHUNK,), jnp.int32),
            pltpu.VMEM((CHUNK,), jnp.int32),
            pltpu.VMEM((EPW,), jnp.float32),
            pltpu.SemaphoreType.DMA,
            pltpu.SemaphoreType.DMA,
        ],
    )(_gather_body)
    return kern(g, src, dst)


def kernel(h, edge_index):
    hb = h.astype(jnp.bfloat16)
    x = jnp.zeros((N_PAD, D_FEAT), jnp.bfloat16).at[:N_NODES].set(hb)
    y = x.T
    g = _gram(x, y)
    out = _gather_flat(g, edge_index[0], edge_index[1])
    return out.reshape(N_EDGES, 1)


# speedup vs baseline: 4.8074x; 4.8074x over previous
import functools

import jax
import jax.numpy as jnp
from jax import lax
from jax.experimental import pallas as pl
from jax.experimental.pallas import tpu as pltpu
from jax.experimental.pallas import tpu_sc as plsc

N_NODES = 10000
N_EDGES = 320000
D_FEAT = 128
LANES = 16
NUM_WORKERS = 32
EPW = N_EDGES // NUM_WORKERS
CHUNK = 80
NCHUNKS = EPW // CHUNK
GROUPS = CHUNK // LANES


def _body(h_hbm, src_hbm, dst_hbm, out_hbm,
          idx_s, idx_d, rows_s0, rows_d0, rows_s1, rows_d1, tbuf, out_v,
          sem0, sem1):
    c = lax.axis_index("c")
    s = lax.axis_index("s")
    wid = s * 2 + c
    base = wid * EPW

    pltpu.sync_copy(src_hbm.at[pl.ds(base, EPW)], idx_s)
    pltpu.sync_copy(dst_hbm.at[pl.ds(base, EPW)], idx_d)

    bufs = ((rows_s0, rows_d0, sem0), (rows_s1, rows_d1, sem1))
    row_iota = lax.iota(jnp.int32, LANES)

    def fire(g, rs, rd, sem):
        off = g * CHUNK
        pltpu.async_copy(h_hbm.at[idx_s.at[pl.ds(off, CHUNK)]], rs, sem)
        pltpu.async_copy(h_hbm.at[idx_d.at[pl.ds(off, CHUNK)]], rd, sem)

    def drain(rs, rd, sem):
        pltpu.make_async_copy(h_hbm.at[pl.ds(0, CHUNK)], rs, sem).wait()
        pltpu.make_async_copy(h_hbm.at[pl.ds(0, CHUNK)], rd, sem).wait()

    def compute(g, rs, rd):
        off = g * CHUNK

        def group_body(grp, carry):
            g16 = grp * LANES
            for k in range(LANES):
                i = g16 + k
                accb = None
                for j in range(D_FEAT // (2 * LANES)):
                    a = plsc.bitcast(rs[i, pl.ds(j * LANES, LANES)], jnp.bfloat16)
                    b = plsc.bitcast(rd[i, pl.ds(j * LANES, LANES)], jnp.bfloat16)
                    p = a * b
                    accb = p if accb is None else accb + p
                lo, hi = plsc.unpack(accb, format=plsc.PackFormat.INTERLEAVED,
                                     preferred_element_type=jnp.float32)
                tbuf[k, pl.ds(0, LANES)] = lo + hi
            res = plsc.load_gather(tbuf, [row_iota, jnp.zeros((LANES,), jnp.int32)])
            for j in range(1, LANES):
                res = res + plsc.load_gather(
                    tbuf, [row_iota, jnp.full((LANES,), j, jnp.int32)])
            out_v[pl.ds(off + g16, LANES)] = res
            return carry

        lax.fori_loop(0, GROUPS, group_body, 0)

    fire(0, *bufs[0])
    fire(1, *bufs[1])

    def outer(g2, carry):
        for b in range(2):
            g = g2 * 2 + b
            rs, rd, sem = bufs[b]
            drain(rs, rd, sem)
            compute(g, rs, rd)

            @pl.when(g + 2 < NCHUNKS)
            def _():
                fire(g + 2, rs, rd, sem)
        return carry

    lax.fori_loop(0, (NCHUNKS - 1) // 2, outer, 0)
    rs, rd, sem = bufs[0]
    drain(rs, rd, sem)
    compute(NCHUNKS - 1, rs, rd)

    pltpu.sync_copy(out_v, out_hbm.at[pl.ds(base, EPW)])


@jax.jit
def _gather_dot(h, src, dst):
    mesh = plsc.VectorSubcoreMesh(core_axis_name="c", subcore_axis_name="s")
    kern = functools.partial(
        pl.kernel,
        mesh=mesh,
        compiler_params=pltpu.CompilerParams(needs_layout_passes=False,
                                             use_tc_tiling_on_sc=False),
        out_type=jax.ShapeDtypeStruct((N_EDGES,), jnp.float32),
        scratch_types=[
            pltpu.VMEM((EPW,), jnp.int32),
            pltpu.VMEM((EPW,), jnp.int32),
            pltpu.VMEM((CHUNK, D_FEAT // 2), jnp.int32),
            pltpu.VMEM((CHUNK, D_FEAT // 2), jnp.int32),
            pltpu.VMEM((CHUNK, D_FEAT // 2), jnp.int32),
            pltpu.VMEM((CHUNK, D_FEAT // 2), jnp.int32),
            pltpu.VMEM((LANES, LANES), jnp.float32),
            pltpu.VMEM((EPW,), jnp.float32),
            pltpu.SemaphoreType.DMA,
            pltpu.SemaphoreType.DMA,
        ],
    )(_body)
    return kern(h, src, dst)


def kernel(h, edge_index):
    h_pairs = lax.bitcast_convert_type(
        h.astype(jnp.bfloat16).reshape(N_NODES, D_FEAT // 2, 2), jnp.int32)
    out = _gather_dot(h_pairs, edge_index[0], edge_index[1])
    return out.reshape(N_EDGES, 1)


# speedup vs baseline: 5.1543x; 1.0722x over previous
import functools

import jax
import jax.numpy as jnp
from jax import lax
from jax.experimental import pallas as pl
from jax.experimental.pallas import tpu as pltpu
from jax.experimental.pallas import tpu_sc as plsc

N_NODES = 10000
N_EDGES = 320000
D_FEAT = 128
LANES = 16
NUM_WORKERS = 32
EPW = N_EDGES // NUM_WORKERS
CHUNK = 80
NCHUNKS = EPW // CHUNK
GROUPS = CHUNK // LANES


def _body(h_hbm, src_hbm, dst_hbm, out_hbm,
          idx_s, idx_d, rows_s0, rows_d0, rows_s1, rows_d1, tbuf, out_v,
          sem0, sem1):
    c = lax.axis_index("c")
    s = lax.axis_index("s")
    wid = s * 2 + c
    base = wid * EPW

    pltpu.sync_copy(src_hbm.at[pl.ds(base, EPW)], idx_s)
    pltpu.sync_copy(dst_hbm.at[pl.ds(base, EPW)], idx_d)

    bufs = ((rows_s0, rows_d0, sem0), (rows_s1, rows_d1, sem1))
    row_iota = lax.iota(jnp.int32, LANES)

    def fire(g, rs, rd, sem):
        off = g * CHUNK
        pltpu.async_copy(h_hbm.at[idx_s.at[pl.ds(off, CHUNK)]], rs, sem)
        pltpu.async_copy(h_hbm.at[idx_d.at[pl.ds(off, CHUNK)]], rd, sem)

    def drain(rs, rd, sem):
        pltpu.make_async_copy(h_hbm.at[pl.ds(0, CHUNK)], rs, sem).wait()
        pltpu.make_async_copy(h_hbm.at[pl.ds(0, CHUNK)], rd, sem).wait()

    def compute(g, rs, rd):
        off = g * CHUNK

        def group_body(grp, carry):
            g16 = grp * LANES
            for k in range(LANES):
                i = g16 + k
                accb = None
                for j in range(D_FEAT // (2 * LANES)):
                    a = plsc.bitcast(rs[i, pl.ds(j * LANES, LANES)], jnp.bfloat16)
                    b = plsc.bitcast(rd[i, pl.ds(j * LANES, LANES)], jnp.bfloat16)
                    p = a * b
                    accb = p if accb is None else accb + p
                tbuf[k, pl.ds(0, LANES)] = plsc.bitcast(accb, jnp.int32)
            res = None
            for q in range(LANES // 4):
                sb = None
                for j in range(q * 4, q * 4 + 4):
                    v = plsc.load_gather(
                        tbuf, [row_iota, jnp.full((LANES,), j, jnp.int32)])
                    vb = plsc.bitcast(v, jnp.bfloat16)
                    sb = vb if sb is None else sb + vb
                lo, hi = plsc.unpack(sb, format=plsc.PackFormat.INTERLEAVED,
                                     preferred_element_type=jnp.float32)
                t = lo + hi
                res = t if res is None else res + t
            out_v[pl.ds(off + g16, LANES)] = res
            return carry

        lax.fori_loop(0, GROUPS, group_body, 0)

    fire(0, *bufs[0])
    fire(1, *bufs[1])

    def outer(g2, carry):
        for b in range(2):
            g = g2 * 2 + b
            rs, rd, sem = bufs[b]
            drain(rs, rd, sem)
            compute(g, rs, rd)

            @pl.when(g + 2 < NCHUNKS)
            def _():
                fire(g + 2, rs, rd, sem)
        return carry

    lax.fori_loop(0, (NCHUNKS - 1) // 2, outer, 0)
    rs, rd, sem = bufs[0]
    drain(rs, rd, sem)
    compute(NCHUNKS - 1, rs, rd)

    pltpu.sync_copy(out_v, out_hbm.at[pl.ds(base, EPW)])


@jax.jit
def _gather_dot(h, src, dst):
    mesh = plsc.VectorSubcoreMesh(core_axis_name="c", subcore_axis_name="s")
    kern = functools.partial(
        pl.kernel,
        mesh=mesh,
        compiler_params=pltpu.CompilerParams(needs_layout_passes=False,
                                             use_tc_tiling_on_sc=False),
        out_type=jax.ShapeDtypeStruct((N_EDGES,), jnp.float32),
        scratch_types=[
            pltpu.VMEM((EPW,), jnp.int32),
            pltpu.VMEM((EPW,), jnp.int32),
            pltpu.VMEM((CHUNK, D_FEAT // 2), jnp.int32),
            pltpu.VMEM((CHUNK, D_FEAT // 2), jnp.int32),
            pltpu.VMEM((CHUNK, D_FEAT // 2), jnp.int32),
            pltpu.VMEM((CHUNK, D_FEAT // 2), jnp.int32),
            pltpu.VMEM((LANES, LANES), jnp.int32),
            pltpu.VMEM((EPW,), jnp.float32),
            pltpu.SemaphoreType.DMA,
            pltpu.SemaphoreType.DMA,
        ],
    )(_body)
    return kern(h, src, dst)


def kernel(h, edge_index):
    h_pairs = lax.bitcast_convert_type(
        h.astype(jnp.bfloat16).reshape(N_NODES, D_FEAT // 2, 2), jnp.int32)
    out = _gather_dot(h_pairs, edge_index[0], edge_index[1])
    return out.reshape(N_EDGES, 1)
